# baseline (speedup 1.0000x reference)
import functools

import jax
import jax.numpy as jnp
from jax import lax
from jax.experimental import pallas as pl
from jax.experimental.pallas import tpu as pltpu
from jax.experimental.pallas import tpu_sc as plsc

D_MODEL = 64
SCALE = 8.0

_INFO = plsc.get_sparse_core_info()
NC = _INFO.num_cores
NS = _INFO.num_subcores
NW = NC * NS
LANES = _INFO.num_lanes

CHUNK = 128


def _make_kernel(n_idx: int):
  assert n_idx % (NW * CHUNK) == 0
  per_w = n_idx // NW
  n_chunks = per_w // CHUNK

  mesh = plsc.VectorSubcoreMesh(core_axis_name="c", subcore_axis_name="s")

  @functools.partial(
      pl.kernel,
      out_type=jax.ShapeDtypeStruct((n_idx, D_MODEL), jnp.float32),
      mesh=mesh,
      scratch_types=[
          pltpu.VMEM((n_chunks, CHUNK), jnp.int32),
          pltpu.VMEM((CHUNK, D_MODEL), jnp.float32),
          pltpu.SemaphoreType.DMA,
          pltpu.SemaphoreType.DMA,
      ],
      compiler_params=pltpu.CompilerParams(use_tc_tiling_on_sc=False),
  )
  def emb_kernel(idx_hbm, table_hbm, out_hbm, idx_v, rows_v, gsem, psem):
    wid = lax.axis_index("s") * NC + lax.axis_index("c")
    base = wid * per_w
    pltpu.sync_copy(idx_hbm.at[wid], idx_v)

    @pl.loop(0, n_chunks)
    def _chunk(j):
      pltpu.async_copy(table_hbm.at[idx_v.at[j]], rows_v, gsem).wait()

      @pl.loop(0, CHUNK, unroll=4)
      def _row(r):
        for c in range(D_MODEL // LANES):
          sl = pl.ds(c * LANES, LANES)
          rows_v[r, sl] = rows_v[r, sl] * SCALE

      pltpu.async_copy(
          rows_v, out_hbm.at[pl.ds(base + j * CHUNK, CHUNK)], psem
      ).wait()

  return emb_kernel


def kernel(x, table):
  b, s = x.shape
  n_idx = b * s
  idx = x.reshape(NW, n_idx // (NW * CHUNK), CHUNK).astype(jnp.int32)
  out = _make_kernel(n_idx)(idx, table)
  return out.reshape(b, s, D_MODEL)


# speedup vs baseline: 1.1652x; 1.1652x over previous
import functools

import jax
import jax.numpy as jnp
from jax import lax
from jax.experimental import pallas as pl
from jax.experimental.pallas import tpu as pltpu
from jax.experimental.pallas import tpu_sc as plsc

D_MODEL = 64
SCALE = 8.0

_INFO = plsc.get_sparse_core_info()
NC = _INFO.num_cores
NS = _INFO.num_subcores
NW = NC * NS
LANES = _INFO.num_lanes

CHUNK = 128
NBUF = 8
HALF = NBUF // 2


def _make_kernel(n_idx: int):
  assert n_idx % (NW * CHUNK) == 0
  per_w = n_idx // NW
  n_chunks = per_w // CHUNK
  assert n_chunks > NBUF

  mesh = plsc.VectorSubcoreMesh(core_axis_name="c", subcore_axis_name="s")

  @functools.partial(
      pl.kernel,
      out_type=jax.ShapeDtypeStruct((n_idx, D_MODEL), jnp.float32),
      mesh=mesh,
      scratch_types=[
          pltpu.VMEM((n_chunks, CHUNK), jnp.int32),
          pltpu.VMEM((NBUF, CHUNK, D_MODEL), jnp.float32),
          pltpu.SemaphoreType.DMA,
          pltpu.SemaphoreType.DMA,
      ],
      compiler_params=pltpu.CompilerParams(use_tc_tiling_on_sc=False),
  )
  def emb_kernel(idx_hbm, table_hbm, out_hbm, idx_v, rows_v, gsem, psem):
    wid = lax.axis_index("s") * NC + lax.axis_index("c")
    base = wid * per_w
    pltpu.sync_copy(idx_hbm.at[wid], idx_v)

    def fire_gather(j):
      pltpu.async_copy(table_hbm.at[idx_v.at[j]], rows_v.at[j % NBUF], gsem)

    def wait_one(sem):
      pltpu.make_async_copy(rows_v.at[0], out_hbm.at[pl.ds(0, CHUNK)],
                            sem).wait()

    for j in range(HALF):
      fire_gather(j)

    @pl.loop(0, n_chunks)
    def _chunk(j):
      bi = j % NBUF
      wait_one(gsem)

      @pl.loop(0, CHUNK, unroll=4)
      def _row(r):
        for c in range(D_MODEL // LANES):
          sl = pl.ds(c * LANES, LANES)
          rows_v[bi, r, sl] = rows_v[bi, r, sl] * SCALE

      pltpu.async_copy(
          rows_v.at[bi], out_hbm.at[pl.ds(base + j * CHUNK, CHUNK)], psem
      )

      jn = j + HALF

      @pl.when(jn < n_chunks)
      def _():
        @pl.when(j >= HALF)
        def _():
          wait_one(psem)
        fire_gather(jn)

    @pl.loop(0, NBUF)
    def _drain(_):
      wait_one(psem)

  return emb_kernel


def kernel(x, table):
  b, s = x.shape
  n_idx = b * s
  idx = x.reshape(NW, n_idx // (NW * CHUNK), CHUNK).astype(jnp.int32)
  out = _make_kernel(n_idx)(idx, table)
  return out.reshape(b, s, D_MODEL)
